# Initial kernel scaffold; baseline (speedup 1.0000x reference)
import jax
import jax.numpy as jnp
from jax.experimental import pallas as pl


def kernel(h, edge_index_r0, edge_index_r1, ew_r0, ew_r1, W0, b0, W1, b1, A1, ab1, A2):
    raise NotImplementedError("write your pallas kernel here")


# speedup vs baseline: 3.5973x; 3.5973x over previous
import dataclasses
import functools

import jax
import jax.numpy as jnp
from jax import lax
from jax.experimental import pallas as pl
from jax.experimental.pallas import tpu as pltpu
from jax.experimental.pallas import tpu_sc as plsc

_NC = 2
_NS = 16
_LANES = 16
_CHUNK = 80


def _sc_aggregate(h, src0, dst0, ew0, src1, dst1, ew1, zeros_acc):
    N, D = h.shape
    E = ew0.shape[0]
    n_chunks = E // _CHUNK
    row_blk = 80
    n_row_blks = N // row_blk
    mesh = plsc.VectorSubcoreMesh(
        core_axis_name="c", subcore_axis_name="s", num_cores=_NC, num_subcores=_NS
    )
    cp = pltpu.CompilerParams()
    if "needs_layout_passes" in pltpu.CompilerParams.__dataclass_fields__:
        cp = dataclasses.replace(cp, needs_layout_passes=False)

    @functools.partial(
        pl.kernel,
        out_type=(
            jax.ShapeDtypeStruct((N, D), jnp.float32),
            jax.ShapeDtypeStruct((N, D), jnp.float32),
            jax.ShapeDtypeStruct((_NS * N,), jnp.float32),
            jax.ShapeDtypeStruct((_NS * N,), jnp.float32),
        ),
        mesh=mesh,
        scratch_types=[
            pltpu.VMEM_SHARED((N, D), jnp.float32),
            pltpu.VMEM((_CHUNK,), jnp.int32),
            pltpu.VMEM((_CHUNK,), jnp.int32),
            pltpu.VMEM((_CHUNK,), jnp.float32),
            pltpu.VMEM((_CHUNK, D), jnp.float32),
            pltpu.VMEM((N,), jnp.float32),
            pltpu.SemaphoreType.DMA,
        ],
        compiler_params=cp,
    )
    def k(h_hbm, src0_hbm, dst0_hbm, ew0_hbm, src1_hbm, dst1_hbm, ew1_hbm,
          za_hbm, agg0_hbm, agg1_hbm, degp0_hbm, degp1_hbm,
          acc_sh, src_v, dst_v, ew_v, rows_v, cnt_v, sem):
        c = lax.axis_index("c")
        s = lax.axis_index("s")
        zero16 = jnp.zeros((_LANES,), jnp.float32)
        one16 = jnp.ones((_LANES,), jnp.float32)

        @pl.loop(0, N, step=_LANES)
        def _(i):
            cnt_v.at[pl.ds(i, _LANES)][...] = zero16

        @pl.loop(s, n_row_blks, step=_NS)
        def _(g):
            r0 = g * row_blk
            pltpu.sync_copy(za_hbm.at[pl.ds(r0, row_blk)],
                            acc_sh.at[pl.ds(r0, row_blk)])

        plsc.subcore_barrier()

        def do_relation(src_hbm, dst_hbm, ew_hbm, degp_hbm):
            @pl.loop(s, n_chunks, step=_NS)
            def _(kc):
                off = kc * _CHUNK
                pltpu.sync_copy(src_hbm.at[pl.ds(off, _CHUNK)], src_v)
                pltpu.sync_copy(dst_hbm.at[pl.ds(off, _CHUNK)], dst_v)
                pltpu.sync_copy(ew_hbm.at[pl.ds(off, _CHUNK)], ew_v)
                pltpu.async_copy(h_hbm.at[src_v], rows_v, sem).wait()

                @pl.loop(0, _CHUNK)
                def _(e):
                    w = plsc.load_gather(ew_v, [jnp.full((_LANES,), e, jnp.int32)])
                    for j in range(D // _LANES):
                        sl = (e, pl.ds(j * _LANES, _LANES))
                        rows_v.at[sl][...] = rows_v.at[sl][...] * w

                pltpu.sync_copy(rows_v, acc_sh.at[dst_v], add=True)

                for g in range(_CHUNK // _LANES):
                    d16 = dst_v.at[pl.ds(g * _LANES, _LANES)][...]
                    plsc.addupdate_scatter(cnt_v, [d16], one16)

            pltpu.sync_copy(cnt_v, degp_hbm.at[pl.ds(s * N, N)])

        @pl.when(c == 0)
        def _():
            do_relation(src0_hbm, dst0_hbm, ew0_hbm, degp0_hbm)

        @pl.when(c == 1)
        def _():
            do_relation(src1_hbm, dst1_hbm, ew1_hbm, degp1_hbm)

        plsc.subcore_barrier()

        def copy_out(agg_hbm):
            @pl.loop(s, n_row_blks, step=_NS)
            def _(g):
                r0 = g * row_blk
                pltpu.sync_copy(acc_sh.at[pl.ds(r0, row_blk)],
                                agg_hbm.at[pl.ds(r0, row_blk)])

        @pl.when(c == 0)
        def _():
            copy_out(agg0_hbm)

        @pl.when(c == 1)
        def _():
            copy_out(agg1_hbm)

    return k(h, src0, dst0, ew0, src1, dst1, ew1, zeros_acc)


_ROW_BLK = 2000


def _tc_finish(a0, a1, g0, g1, W0, b0, W1, b1, A1, ab1, A2):
    N, D = a0.shape
    B = _ROW_BLK
    nb = N // B
    hp = lax.Precision.HIGHEST

    def body1(a0_r, a1_r, g0_r, g1_r, w0_r, c0_r, w1_r, c1_r, am_r, ab_r, a2_r,
              x0_o, x1_o, s0_o, s1_o):
        i = pl.program_id(0)
        d0 = jnp.maximum(jnp.sum(g0_r[...], axis=1, keepdims=True), 1.0)
        d1 = jnp.maximum(jnp.sum(g1_r[...], axis=1, keepdims=True), 1.0)
        x0 = jnp.dot(a0_r[...], w0_r[...], precision=hp,
                     preferred_element_type=jnp.float32) / d0 + c0_r[...]
        x1 = jnp.dot(a1_r[...], w1_r[...], precision=hp,
                     preferred_element_type=jnp.float32) / d1 + c1_r[...]
        x0 = jnp.where(x0 > 0, x0, jnp.exp(jnp.minimum(x0, 0.0)) - 1.0)
        x1 = jnp.where(x1 > 0, x1, jnp.exp(jnp.minimum(x1, 0.0)) - 1.0)
        x0_o[...] = x0
        x1_o[...] = x1
        t0 = jnp.tanh(jnp.dot(x0, am_r[...], precision=hp,
                              preferred_element_type=jnp.float32) + ab_r[...])
        t1 = jnp.tanh(jnp.dot(x1, am_r[...], precision=hp,
                              preferred_element_type=jnp.float32) + ab_r[...])
        p0 = jnp.sum(t0 * a2_r[...])
        p1 = jnp.sum(t1 * a2_r[...])

        @pl.when(i == 0)
        def _():
            s0_o[...] = jnp.zeros_like(s0_o)
            s1_o[...] = jnp.zeros_like(s1_o)

        s0_o[...] += p0
        s1_o[...] += p1

    row_spec = pl.BlockSpec((B, D), lambda i: (i, 0))
    deg_spec = pl.BlockSpec((B, _NS), lambda i: (i, 0))
    full_spec = pl.BlockSpec((D, D), lambda i: (0, 0))
    vec_spec = pl.BlockSpec((1, D), lambda i: (0, 0))
    x0, x1, ssum0, ssum1 = pl.pallas_call(
        body1,
        grid=(nb,),
        in_specs=[row_spec, row_spec, deg_spec, deg_spec,
                  full_spec, vec_spec, full_spec, vec_spec,
                  full_spec, vec_spec, vec_spec],
        out_specs=[row_spec, row_spec,
                   pl.BlockSpec((1, D), lambda i: (0, 0)),
                   pl.BlockSpec((1, D), lambda i: (0, 0))],
        out_shape=[jax.ShapeDtypeStruct((N, D), jnp.float32),
                   jax.ShapeDtypeStruct((N, D), jnp.float32),
                   jax.ShapeDtypeStruct((1, D), jnp.float32),
                   jax.ShapeDtypeStruct((1, D), jnp.float32)],
    )(a0, a1, g0, g1, W0, b0, W1, b1, A1, ab1, A2)

    def body2(x0_r, x1_r, s0_r, s1_r, out_r):
        w0m = s0_r[0, 0] / N
        w1m = s1_r[0, 0] / N
        m = jnp.maximum(w0m, w1m)
        e0 = jnp.exp(w0m - m)
        e1 = jnp.exp(w1m - m)
        beta0 = e0 / (e0 + e1)
        beta1 = e1 / (e0 + e1)
        out_r[...] = beta0 * x0_r[...] + beta1 * x1_r[...]

    return pl.pallas_call(
        body2,
        grid=(nb,),
        in_specs=[row_spec, row_spec, vec_spec, vec_spec],
        out_specs=row_spec,
        out_shape=jax.ShapeDtypeStruct((N, D), jnp.float32),
    )(x0, x1, ssum0, ssum1)


def kernel(h, edge_index_r0, edge_index_r1, ew_r0, ew_r1,
           W0, b0, W1, b1, A1, ab1, A2):
    N, D = h.shape
    zeros_acc = jnp.zeros((N, D), jnp.float32)
    agg0, agg1, degp0, degp1 = _sc_aggregate(
        h, edge_index_r0[0], edge_index_r0[1], ew_r0,
        edge_index_r1[0], edge_index_r1[1], ew_r1, zeros_acc)
    g0 = degp0.reshape(_NS, N).T
    g1 = degp1.reshape(_NS, N).T
    return _tc_finish(
        agg0, agg1, g0, g1,
        W0, b0.reshape(1, D), W1, b1.reshape(1, D),
        A1, ab1.reshape(1, -1), A2.reshape(1, -1),
    )


# speedup vs baseline: 3.9191x; 1.0895x over previous
import dataclasses
import functools

import jax
import jax.numpy as jnp
from jax import lax
from jax.experimental import pallas as pl
from jax.experimental.pallas import tpu as pltpu
from jax.experimental.pallas import tpu_sc as plsc

_NC = 2
_NS = 16
_LANES = 16
_CHUNK = 40
_META = 128


def _pack_meta(src, ew, dst):
    nch = src.shape[0] // _CHUNK
    s2 = src.reshape(nch, _CHUNK)
    e2 = jax.lax.bitcast_convert_type(ew, jnp.int32).reshape(nch, _CHUNK)
    d2 = dst.reshape(nch, _CHUNK)
    pad = jnp.zeros((nch, _META - 3 * _CHUNK), jnp.int32)
    return jnp.concatenate([s2, e2, d2, pad], axis=1).reshape(-1)


def _sc_aggregate(h, meta0, dst0, meta1, dst1, zeros_acc, n_edges):
    N, D = h.shape
    E = n_edges
    n_chunks = E // _CHUNK
    per_sub = n_chunks // _NS
    n_pairs = per_sub // 2
    row_blk = 80
    n_row_blks = N // row_blk
    mesh = plsc.VectorSubcoreMesh(
        core_axis_name="c", subcore_axis_name="s", num_cores=_NC, num_subcores=_NS
    )
    cp = pltpu.CompilerParams()
    if "needs_layout_passes" in pltpu.CompilerParams.__dataclass_fields__:
        cp = dataclasses.replace(cp, needs_layout_passes=False)

    @functools.partial(
        pl.kernel,
        out_type=(
            jax.ShapeDtypeStruct((N, D), jnp.float32),
            jax.ShapeDtypeStruct((N, D), jnp.float32),
            jax.ShapeDtypeStruct((_NS * N,), jnp.float32),
            jax.ShapeDtypeStruct((_NS * N,), jnp.float32),
        ),
        mesh=mesh,
        scratch_types=[
            pltpu.VMEM_SHARED((N, D), jnp.float32),
            pltpu.VMEM((_META,), jnp.int32),
            pltpu.VMEM((_META,), jnp.int32),
            pltpu.VMEM((_CHUNK,), jnp.int32),
            pltpu.VMEM((_CHUNK,), jnp.int32),
            pltpu.VMEM((_CHUNK, D), jnp.float32),
            pltpu.VMEM((_CHUNK, D), jnp.float32),
            pltpu.VMEM((N,), jnp.float32),
            pltpu.SemaphoreType.DMA,
            pltpu.SemaphoreType.DMA,
            pltpu.SemaphoreType.DMA,
            pltpu.SemaphoreType.DMA,
        ],
        compiler_params=cp,
    )
    def k(h_hbm, meta0_hbm, dst0_hbm, meta1_hbm, dst1_hbm,
          za_hbm, agg0_hbm, agg1_hbm, degp0_hbm, degp1_hbm,
          acc_sh, cb0, cb1, db0, db1, rows0, rows1, cnt_v,
          sem_g0, sem_g1, sem_s0, sem_s1):
        c = lax.axis_index("c")
        s = lax.axis_index("s")
        zero16 = jnp.zeros((_LANES,), jnp.float32)
        one16 = jnp.ones((_LANES,), jnp.float32)
        iota16 = lax.broadcasted_iota(jnp.int32, (_LANES,), 0)
        tail_mask = iota16 < (3 * _CHUNK - 112)

        @pl.loop(0, N, step=_LANES)
        def _(i):
            cnt_v.at[pl.ds(i, _LANES)][...] = zero16

        @pl.loop(s, n_row_blks, step=_NS)
        def _(g):
            r0 = g * row_blk
            pltpu.sync_copy(za_hbm.at[pl.ds(r0, row_blk)],
                            acc_sh.at[pl.ds(r0, row_blk)])

        plsc.subcore_barrier()

        dummy = za_hbm.at[pl.ds(0, _CHUNK)]

        def do_relation(meta_hbm, dst_hbm, degp_hbm):
            bufs = ((cb0, db0, rows0, sem_g0, sem_s0),
                    (cb1, db1, rows1, sem_g1, sem_s1))

            def issue(j, p, guard_drain):
                cb, db, rb, sg, ss = bufs[p]
                kc = s + j * _NS

                def drain():
                    pltpu.make_async_copy(dummy, rb, ss).wait()

                if guard_drain is None:
                    drain()
                else:
                    pl.when(guard_drain)(drain)
                pltpu.sync_copy(meta_hbm.at[pl.ds(kc * _META, _META)], cb)
                pltpu.sync_copy(dst_hbm.at[pl.ds(kc * _CHUNK, _CHUNK)], db)
                pltpu.async_copy(h_hbm.at[cb.at[pl.ds(0, _CHUNK)]], rb, sg)

            def process(p):
                cb, db, rb, sg, ss = bufs[p]
                pltpu.make_async_copy(dummy, rb, sg).wait()

                @pl.loop(0, _CHUNK)
                def _(e):
                    wbits = plsc.load_gather(
                        cb, [jnp.full((_LANES,), _CHUNK + e, jnp.int32)])
                    w = plsc.bitcast(wbits, jnp.float32)
                    for jj in range(D // _LANES):
                        sl = (e, pl.ds(jj * _LANES, _LANES))
                        rb.at[sl][...] = rb.at[sl][...] * w

                full_groups = _CHUNK // _LANES
                for gi in range(full_groups):
                    d16 = cb.at[pl.ds(2 * _CHUNK + gi * _LANES, _LANES)][...]
                    plsc.addupdate_scatter(cnt_v, [d16], one16)
                rem = _CHUNK % _LANES
                if rem:
                    d16 = cb.at[pl.ds(2 * _CHUNK + full_groups * _LANES,
                                      _LANES)][...]
                    plsc.addupdate_scatter(cnt_v, [d16], one16,
                                           mask=iota16 < rem)
                pltpu.async_copy(rb, acc_sh.at[db], sem=ss, add=True)

            @pl.loop(0, n_pairs)
            def _(u):
                issue(2 * u, 0, u >= 1)
                pl.when(u >= 1)(lambda: process(1))
                issue(2 * u + 1, 1, u >= 1)
                process(0)

            process(1)
            pltpu.make_async_copy(dummy, rows0, sem_s0).wait()
            pltpu.make_async_copy(dummy, rows1, sem_s1).wait()
            pltpu.sync_copy(cnt_v, degp_hbm.at[pl.ds(s * N, N)])

        @pl.when(c == 0)
        def _():
            do_relation(meta0_hbm, dst0_hbm, degp0_hbm)

        @pl.when(c == 1)
        def _():
            do_relation(meta1_hbm, dst1_hbm, degp1_hbm)

        plsc.subcore_barrier()

        def copy_out(agg_hbm):
            @pl.loop(s, n_row_blks, step=_NS)
            def _(g):
                r0 = g * row_blk
                pltpu.sync_copy(acc_sh.at[pl.ds(r0, row_blk)],
                                agg_hbm.at[pl.ds(r0, row_blk)])

        @pl.when(c == 0)
        def _():
            copy_out(agg0_hbm)

        @pl.when(c == 1)
        def _():
            copy_out(agg1_hbm)

    return k(h, meta0, dst0, meta1, dst1, zeros_acc)


_ROW_BLK = 2000


def _tc_finish(a0, a1, g0, g1, W0, b0, W1, b1, A1, ab1, A2):
    N, D = a0.shape
    B = _ROW_BLK
    nb = N // B
    hp = lax.Precision.HIGHEST

    def body1(a0_r, a1_r, g0_r, g1_r, w0_r, c0_r, w1_r, c1_r, am_r, ab_r, a2_r,
              x0_o, x1_o, s0_o, s1_o):
        i = pl.program_id(0)
        d0 = jnp.maximum(jnp.sum(g0_r[...], axis=1, keepdims=True), 1.0)
        d1 = jnp.maximum(jnp.sum(g1_r[...], axis=1, keepdims=True), 1.0)
        x0 = jnp.dot(a0_r[...], w0_r[...], precision=hp,
                     preferred_element_type=jnp.float32) / d0 + c0_r[...]
        x1 = jnp.dot(a1_r[...], w1_r[...], precision=hp,
                     preferred_element_type=jnp.float32) / d1 + c1_r[...]
        x0 = jnp.where(x0 > 0, x0, jnp.exp(jnp.minimum(x0, 0.0)) - 1.0)
        x1 = jnp.where(x1 > 0, x1, jnp.exp(jnp.minimum(x1, 0.0)) - 1.0)
        x0_o[...] = x0
        x1_o[...] = x1
        t0 = jnp.tanh(jnp.dot(x0, am_r[...], precision=hp,
                              preferred_element_type=jnp.float32) + ab_r[...])
        t1 = jnp.tanh(jnp.dot(x1, am_r[...], precision=hp,
                              preferred_element_type=jnp.float32) + ab_r[...])
        p0 = jnp.sum(t0 * a2_r[...])
        p1 = jnp.sum(t1 * a2_r[...])

        @pl.when(i == 0)
        def _():
            s0_o[...] = jnp.zeros_like(s0_o)
            s1_o[...] = jnp.zeros_like(s1_o)

        s0_o[...] += p0
        s1_o[...] += p1

    row_spec = pl.BlockSpec((B, D), lambda i: (i, 0))
    deg_spec = pl.BlockSpec((B, _NS), lambda i: (i, 0))
    full_spec = pl.BlockSpec((D, D), lambda i: (0, 0))
    vec_spec = pl.BlockSpec((1, D), lambda i: (0, 0))
    x0, x1, ssum0, ssum1 = pl.pallas_call(
        body1,
        grid=(nb,),
        in_specs=[row_spec, row_spec, deg_spec, deg_spec,
                  full_spec, vec_spec, full_spec, vec_spec,
                  full_spec, vec_spec, vec_spec],
        out_specs=[row_spec, row_spec,
                   pl.BlockSpec((1, D), lambda i: (0, 0)),
                   pl.BlockSpec((1, D), lambda i: (0, 0))],
        out_shape=[jax.ShapeDtypeStruct((N, D), jnp.float32),
                   jax.ShapeDtypeStruct((N, D), jnp.float32),
                   jax.ShapeDtypeStruct((1, D), jnp.float32),
                   jax.ShapeDtypeStruct((1, D), jnp.float32)],
    )(a0, a1, g0, g1, W0, b0, W1, b1, A1, ab1, A2)

    def body2(x0_r, x1_r, s0_r, s1_r, out_r):
        w0m = s0_r[0, 0] / N
        w1m = s1_r[0, 0] / N
        m = jnp.maximum(w0m, w1m)
        e0 = jnp.exp(w0m - m)
        e1 = jnp.exp(w1m - m)
        beta0 = e0 / (e0 + e1)
        beta1 = e1 / (e0 + e1)
        out_r[...] = beta0 * x0_r[...] + beta1 * x1_r[...]

    return pl.pallas_call(
        body2,
        grid=(nb,),
        in_specs=[row_spec, row_spec, vec_spec, vec_spec],
        out_specs=row_spec,
        out_shape=jax.ShapeDtypeStruct((N, D), jnp.float32),
    )(x0, x1, ssum0, ssum1)


def kernel(h, edge_index_r0, edge_index_r1, ew_r0, ew_r1,
           W0, b0, W1, b1, A1, ab1, A2):
    N, D = h.shape
    E = ew_r0.shape[0]
    zeros_acc = jnp.zeros((N, D), jnp.float32)
    meta0 = _pack_meta(edge_index_r0[0], ew_r0, edge_index_r0[1])
    meta1 = _pack_meta(edge_index_r1[0], ew_r1, edge_index_r1[1])
    agg0, agg1, degp0, degp1 = _sc_aggregate(
        h, meta0, edge_index_r0[1], meta1, edge_index_r1[1], zeros_acc, E)
    g0 = degp0.reshape(_NS, N).T
    g1 = degp1.reshape(_NS, N).T
    return _tc_finish(
        agg0, agg1, g0, g1,
        W0, b0.reshape(1, D), W1, b1.reshape(1, D),
        A1, ab1.reshape(1, -1), A2.reshape(1, -1),
    )


# speedup vs baseline: 4.3833x; 1.1184x over previous
import dataclasses
import functools

import jax
import jax.numpy as jnp
from jax import lax
from jax.experimental import pallas as pl
from jax.experimental.pallas import tpu as pltpu
from jax.experimental.pallas import tpu_sc as plsc

_NC = 2
_NS = 16
_LANES = 16
_CHUNK = 40
_META = 128


def _pack_meta(src, ew, dst):
    nch = src.shape[0] // _CHUNK
    s2 = src.reshape(nch, _CHUNK)
    e2 = jax.lax.bitcast_convert_type(ew, jnp.int32).reshape(nch, _CHUNK)
    d2 = dst.reshape(nch, _CHUNK)
    pad = jnp.zeros((nch, _META - 3 * _CHUNK), jnp.int32)
    return jnp.concatenate([s2, e2, d2, pad], axis=1).reshape(-1)


def _sc_aggregate(h, meta0, dst0, meta1, dst1, zeros_acc, n_edges):
    N, D = h.shape
    E = n_edges
    n_chunks = E // _CHUNK
    per_sub = n_chunks // _NS
    n_pairs = per_sub // 2
    row_blk = 80
    n_row_blks = N // row_blk
    mesh = plsc.VectorSubcoreMesh(
        core_axis_name="c", subcore_axis_name="s", num_cores=_NC, num_subcores=_NS
    )
    cp = pltpu.CompilerParams()
    if "needs_layout_passes" in pltpu.CompilerParams.__dataclass_fields__:
        cp = dataclasses.replace(cp, needs_layout_passes=False)

    @functools.partial(
        pl.kernel,
        out_type=(
            jax.ShapeDtypeStruct((N, D), jnp.float32),
            jax.ShapeDtypeStruct((N, D), jnp.float32),
            jax.ShapeDtypeStruct((_NS * N,), jnp.float32),
            jax.ShapeDtypeStruct((_NS * N,), jnp.float32),
        ),
        mesh=mesh,
        scratch_types=[
            pltpu.VMEM_SHARED((N, D), jnp.float32),
            pltpu.VMEM((_META,), jnp.int32),
            pltpu.VMEM((_META,), jnp.int32),
            pltpu.VMEM((_CHUNK,), jnp.int32),
            pltpu.VMEM((_CHUNK,), jnp.int32),
            pltpu.VMEM((_CHUNK, D), jnp.float32),
            pltpu.VMEM((_CHUNK, D), jnp.float32),
            pltpu.VMEM((N,), jnp.float32),
            pltpu.SemaphoreType.DMA,
            pltpu.SemaphoreType.DMA,
            pltpu.SemaphoreType.DMA,
            pltpu.SemaphoreType.DMA,
        ],
        compiler_params=cp,
    )
    def k(h_hbm, meta0_hbm, dst0_hbm, meta1_hbm, dst1_hbm,
          za_hbm, agg0_hbm, agg1_hbm, degp0_hbm, degp1_hbm,
          acc_sh, cb0, cb1, db0, db1, rows0, rows1, cnt_v,
          sem_g0, sem_g1, sem_s0, sem_s1):
        c = lax.axis_index("c")
        s = lax.axis_index("s")
        zero16 = jnp.zeros((_LANES,), jnp.float32)
        one16 = jnp.ones((_LANES,), jnp.float32)
        iota16 = lax.broadcasted_iota(jnp.int32, (_LANES,), 0)
        tail_mask = iota16 < (3 * _CHUNK - 112)

        @pl.loop(0, N, step=_LANES)
        def _(i):
            cnt_v.at[pl.ds(i, _LANES)][...] = zero16

        @pl.loop(s, n_row_blks, step=_NS)
        def _(g):
            r0 = g * row_blk
            pltpu.sync_copy(za_hbm.at[pl.ds(r0, row_blk)],
                            acc_sh.at[pl.ds(r0, row_blk)])

        plsc.subcore_barrier()

        dummy = za_hbm.at[pl.ds(0, _CHUNK)]

        def do_relation(meta_hbm, dst_hbm, degp_hbm):
            bufs = ((cb0, db0, rows0, sem_g0, sem_s0),
                    (cb1, db1, rows1, sem_g1, sem_s1))

            def issue(j, p, guard_drain):
                cb, db, rb, sg, ss = bufs[p]
                kc = s + j * _NS

                def drain():
                    pltpu.make_async_copy(dummy, rb, ss).wait()

                if guard_drain is None:
                    drain()
                else:
                    pl.when(guard_drain)(drain)
                pltpu.sync_copy(meta_hbm.at[pl.ds(kc * _META, _META)], cb)
                pltpu.sync_copy(dst_hbm.at[pl.ds(kc * _CHUNK, _CHUNK)], db)
                pltpu.async_copy(h_hbm.at[cb.at[pl.ds(0, _CHUNK)]], rb, sg)

            def process(p):
                cb, db, rb, sg, ss = bufs[p]
                pltpu.make_async_copy(dummy, rb, sg).wait()

                @plsc.parallel_loop(0, _CHUNK, unroll=4)
                def _(e):
                    wbits = plsc.load_gather(
                        cb, [jnp.full((_LANES,), _CHUNK + e, jnp.int32)])
                    w = plsc.bitcast(wbits, jnp.float32)
                    for jj in range(D // _LANES):
                        sl = (e, pl.ds(jj * _LANES, _LANES))
                        rb.at[sl][...] = rb.at[sl][...] * w

                full_groups = _CHUNK // _LANES
                for gi in range(full_groups):
                    d16 = cb.at[pl.ds(2 * _CHUNK + gi * _LANES, _LANES)][...]
                    plsc.addupdate_scatter(cnt_v, [d16], one16)
                rem = _CHUNK % _LANES
                if rem:
                    d16 = cb.at[pl.ds(2 * _CHUNK + full_groups * _LANES,
                                      _LANES)][...]
                    plsc.addupdate_scatter(cnt_v, [d16], one16,
                                           mask=iota16 < rem)
                pltpu.async_copy(rb, acc_sh.at[db], sem=ss, add=True)

            @pl.loop(0, n_pairs)
            def _(u):
                issue(2 * u, 0, u >= 1)
                pl.when(u >= 1)(lambda: process(1))
                issue(2 * u + 1, 1, u >= 1)
                process(0)

            process(1)
            pltpu.make_async_copy(dummy, rows0, sem_s0).wait()
            pltpu.make_async_copy(dummy, rows1, sem_s1).wait()
            pltpu.sync_copy(cnt_v, degp_hbm.at[pl.ds(s * N, N)])

        @pl.when(c == 0)
        def _():
            do_relation(meta0_hbm, dst0_hbm, degp0_hbm)

        @pl.when(c == 1)
        def _():
            do_relation(meta1_hbm, dst1_hbm, degp1_hbm)

        plsc.subcore_barrier()

        def copy_out(agg_hbm):
            @pl.loop(s, n_row_blks, step=_NS)
            def _(g):
                r0 = g * row_blk
                pltpu.sync_copy(acc_sh.at[pl.ds(r0, row_blk)],
                                agg_hbm.at[pl.ds(r0, row_blk)])

        @pl.when(c == 0)
        def _():
            copy_out(agg0_hbm)

        @pl.when(c == 1)
        def _():
            copy_out(agg1_hbm)

    return k(h, meta0, dst0, meta1, dst1, zeros_acc)


_ROW_BLK = 2000


def _tc_finish(a0, a1, g0, g1, W0, b0, W1, b1, A1, ab1, A2):
    N, D = a0.shape
    B = _ROW_BLK
    nb = N // B
    hp = lax.Precision.HIGHEST

    def body1(a0_r, a1_r, g0_r, g1_r, w0_r, c0_r, w1_r, c1_r, am_r, ab_r, a2_r,
              x0_o, x1_o, s0_o, s1_o):
        i = pl.program_id(0)
        d0 = jnp.maximum(jnp.sum(g0_r[...], axis=1, keepdims=True), 1.0)
        d1 = jnp.maximum(jnp.sum(g1_r[...], axis=1, keepdims=True), 1.0)
        x0 = jnp.dot(a0_r[...], w0_r[...], precision=hp,
                     preferred_element_type=jnp.float32) / d0 + c0_r[...]
        x1 = jnp.dot(a1_r[...], w1_r[...], precision=hp,
                     preferred_element_type=jnp.float32) / d1 + c1_r[...]
        x0 = jnp.where(x0 > 0, x0, jnp.exp(jnp.minimum(x0, 0.0)) - 1.0)
        x1 = jnp.where(x1 > 0, x1, jnp.exp(jnp.minimum(x1, 0.0)) - 1.0)
        x0_o[...] = x0
        x1_o[...] = x1
        t0 = jnp.tanh(jnp.dot(x0, am_r[...], precision=hp,
                              preferred_element_type=jnp.float32) + ab_r[...])
        t1 = jnp.tanh(jnp.dot(x1, am_r[...], precision=hp,
                              preferred_element_type=jnp.float32) + ab_r[...])
        p0 = jnp.sum(t0 * a2_r[...])
        p1 = jnp.sum(t1 * a2_r[...])

        @pl.when(i == 0)
        def _():
            s0_o[...] = jnp.zeros_like(s0_o)
            s1_o[...] = jnp.zeros_like(s1_o)

        s0_o[...] += p0
        s1_o[...] += p1

    row_spec = pl.BlockSpec((B, D), lambda i: (i, 0))
    deg_spec = pl.BlockSpec((B, _NS), lambda i: (i, 0))
    full_spec = pl.BlockSpec((D, D), lambda i: (0, 0))
    vec_spec = pl.BlockSpec((1, D), lambda i: (0, 0))
    x0, x1, ssum0, ssum1 = pl.pallas_call(
        body1,
        grid=(nb,),
        in_specs=[row_spec, row_spec, deg_spec, deg_spec,
                  full_spec, vec_spec, full_spec, vec_spec,
                  full_spec, vec_spec, vec_spec],
        out_specs=[row_spec, row_spec,
                   pl.BlockSpec((1, D), lambda i: (0, 0)),
                   pl.BlockSpec((1, D), lambda i: (0, 0))],
        out_shape=[jax.ShapeDtypeStruct((N, D), jnp.float32),
                   jax.ShapeDtypeStruct((N, D), jnp.float32),
                   jax.ShapeDtypeStruct((1, D), jnp.float32),
                   jax.ShapeDtypeStruct((1, D), jnp.float32)],
    )(a0, a1, g0, g1, W0, b0, W1, b1, A1, ab1, A2)

    def body2(x0_r, x1_r, s0_r, s1_r, out_r):
        w0m = s0_r[0, 0] / N
        w1m = s1_r[0, 0] / N
        m = jnp.maximum(w0m, w1m)
        e0 = jnp.exp(w0m - m)
        e1 = jnp.exp(w1m - m)
        beta0 = e0 / (e0 + e1)
        beta1 = e1 / (e0 + e1)
        out_r[...] = beta0 * x0_r[...] + beta1 * x1_r[...]

    return pl.pallas_call(
        body2,
        grid=(nb,),
        in_specs=[row_spec, row_spec, vec_spec, vec_spec],
        out_specs=row_spec,
        out_shape=jax.ShapeDtypeStruct((N, D), jnp.float32),
    )(x0, x1, ssum0, ssum1)


def kernel(h, edge_index_r0, edge_index_r1, ew_r0, ew_r1,
           W0, b0, W1, b1, A1, ab1, A2):
    N, D = h.shape
    E = ew_r0.shape[0]
    zeros_acc = jnp.zeros((N, D), jnp.float32)
    meta0 = _pack_meta(edge_index_r0[0], ew_r0, edge_index_r0[1])
    meta1 = _pack_meta(edge_index_r1[0], ew_r1, edge_index_r1[1])
    agg0, agg1, degp0, degp1 = _sc_aggregate(
        h, meta0, edge_index_r0[1], meta1, edge_index_r1[1], zeros_acc, E)
    g0 = degp0.reshape(_NS, N).T
    g1 = degp1.reshape(_NS, N).T
    return _tc_finish(
        agg0, agg1, g0, g1,
        W0, b0.reshape(1, D), W1, b1.reshape(1, D),
        A1, ab1.reshape(1, -1), A2.reshape(1, -1),
    )


# speedup vs baseline: 7.4298x; 1.6950x over previous
import dataclasses
import functools

import jax
import jax.numpy as jnp
from jax import lax
from jax.experimental import pallas as pl
from jax.experimental.pallas import tpu as pltpu
from jax.experimental.pallas import tpu_sc as plsc

_NC = 2
_NS = 16
_LANES = 16
_CHUNK = 40
_META = 128
_BLK = 10


def _pack_meta(src, ew, dst):
    nch = src.shape[0] // _CHUNK
    s2 = src.reshape(nch, _CHUNK)
    e2 = jax.lax.bitcast_convert_type(ew, jnp.int32).reshape(nch, _CHUNK)
    d2 = dst.reshape(nch, _CHUNK)
    pad = jnp.zeros((nch, _META - 3 * _CHUNK), jnp.int32)
    return jnp.concatenate([s2, e2, d2, pad], axis=1).reshape(-1)


def _sc_aggregate(h, meta0, dst0, meta1, dst1, zeros_acc, n_edges):
    N, D = h.shape
    E = n_edges
    n_chunks = E // _CHUNK
    per_sub = n_chunks // _NS
    n_blocks = per_sub // _BLK
    row_blk = 80
    n_row_blks = N // row_blk
    mesh = plsc.VectorSubcoreMesh(
        core_axis_name="c", subcore_axis_name="s", num_cores=_NC, num_subcores=_NS
    )
    cp = pltpu.CompilerParams()
    if "needs_layout_passes" in pltpu.CompilerParams.__dataclass_fields__:
        cp = dataclasses.replace(cp, needs_layout_passes=False)

    @functools.partial(
        pl.kernel,
        out_type=(
            jax.ShapeDtypeStruct((N, D), jnp.float32),
            jax.ShapeDtypeStruct((N, D), jnp.float32),
            jax.ShapeDtypeStruct((_NS * N,), jnp.float32),
            jax.ShapeDtypeStruct((_NS * N,), jnp.float32),
        ),
        mesh=mesh,
        scratch_types=[
            pltpu.VMEM_SHARED((N, D), jnp.float32),
            pltpu.VMEM((_BLK * _META,), jnp.int32),
            pltpu.VMEM((_BLK * _META,), jnp.int32),
            pltpu.VMEM((_BLK * _CHUNK,), jnp.int32),
            pltpu.VMEM((_BLK * _CHUNK,), jnp.int32),
            pltpu.VMEM((_CHUNK,), jnp.int32),
            pltpu.VMEM((_CHUNK,), jnp.int32),
            pltpu.VMEM((_CHUNK, D), jnp.float32),
            pltpu.VMEM((_CHUNK, D), jnp.float32),
            pltpu.VMEM((N,), jnp.float32),
            pltpu.SemaphoreType.DMA,
            pltpu.SemaphoreType.DMA,
            pltpu.SemaphoreType.DMA,
            pltpu.SemaphoreType.DMA,
            pltpu.SemaphoreType.DMA,
            pltpu.SemaphoreType.DMA,
        ],
        compiler_params=cp,
    )
    def k(h_hbm, meta0_hbm, dst0_hbm, meta1_hbm, dst1_hbm,
          za_hbm, agg0_hbm, agg1_hbm, degp0_hbm, degp1_hbm,
          acc_sh, mbuf0, mbuf1, dbuf0, dbuf1, dv0, dv1, rows0, rows1, cnt_v,
          sem_g0, sem_g1, sem_s0, sem_s1, sem_m0, sem_m1):
        c = lax.axis_index("c")
        s = lax.axis_index("s")
        zero16 = jnp.zeros((_LANES,), jnp.float32)
        one16 = jnp.ones((_LANES,), jnp.float32)
        iota16 = lax.broadcasted_iota(jnp.int32, (_LANES,), 0)
        tail_mask = iota16 < (3 * _CHUNK - 112)

        @pl.loop(0, N, step=_LANES)
        def _(i):
            cnt_v.at[pl.ds(i, _LANES)][...] = zero16

        @pl.loop(s, n_row_blks, step=_NS)
        def _(g):
            r0 = g * row_blk
            pltpu.sync_copy(za_hbm.at[pl.ds(r0, row_blk)],
                            acc_sh.at[pl.ds(r0, row_blk)])

        plsc.subcore_barrier()

        dummy = za_hbm.at[pl.ds(0, _CHUNK)]

        def do_relation(meta_hbm, dst_hbm, degp_hbm):
            base = s * per_sub
            mbufs = (mbuf0, mbuf1)
            dbufs = (dbuf0, dbuf1)
            cbufs = ((dv0, rows0, sem_g0, sem_s0),
                     (dv1, rows1, sem_g1, sem_s1))
            sem_ms = (sem_m0, sem_m1)

            def load_block(b, bp):
                off = base + b * _BLK
                pltpu.async_copy(meta_hbm.at[pl.ds(off * _META, _BLK * _META)],
                                 mbufs[bp], sem_ms[bp])
                pltpu.async_copy(dst_hbm.at[pl.ds(off * _CHUNK, _BLK * _CHUNK)],
                                 dbufs[bp], sem_ms[bp])

            def wait_block(bp):
                pltpu.make_async_copy(meta_hbm.at[pl.ds(0, _BLK * _META)],
                                      mbufs[bp], sem_ms[bp]).wait()
                pltpu.make_async_copy(dst_hbm.at[pl.ds(0, _BLK * _CHUNK)],
                                      dbufs[bp], sem_ms[bp]).wait()

            def issue(ci, bp, guard_drain):
                p = ci % 2
                dv, rb, sg, ss = cbufs[p]

                def drain():
                    pltpu.make_async_copy(dummy, rb, ss).wait()

                if guard_drain is None:
                    drain()
                else:
                    pl.when(guard_drain)(drain)
                dbf = dbufs[bp]
                for so in (0, _LANES, _CHUNK - _LANES):
                    dv.at[pl.ds(so, _LANES)][...] = \
                        dbf.at[pl.ds(ci * _CHUNK + so, _LANES)][...]
                pltpu.async_copy(
                    h_hbm.at[mbufs[bp].at[pl.ds(ci * _META, _CHUNK)]], rb, sg)

            def process(ci, bp):
                p = ci % 2
                dv, rb, sg, ss = cbufs[p]
                mb = mbufs[bp]
                mo = ci * _META
                pltpu.make_async_copy(dummy, rb, sg).wait()

                @plsc.parallel_loop(0, _CHUNK, unroll=4)
                def _(e):
                    wbits = plsc.load_gather(
                        mb, [jnp.full((_LANES,), mo + _CHUNK + e, jnp.int32)])
                    w = plsc.bitcast(wbits, jnp.float32)
                    for jj in range(D // _LANES):
                        sl = (e, pl.ds(jj * _LANES, _LANES))
                        rb.at[sl][...] = rb.at[sl][...] * w

                full_groups = _CHUNK // _LANES
                for gi in range(full_groups):
                    d16 = mb.at[pl.ds(mo + 2 * _CHUNK + gi * _LANES,
                                      _LANES)][...]
                    plsc.addupdate_scatter(cnt_v, [d16], one16)
                rem = _CHUNK % _LANES
                if rem:
                    d16 = mb.at[pl.ds(mo + 2 * _CHUNK + full_groups * _LANES,
                                      _LANES)][...]
                    plsc.addupdate_scatter(cnt_v, [d16], one16,
                                           mask=iota16 < rem)
                pltpu.async_copy(rb, acc_sh.at[dv], sem=ss, add=True)

            load_block(0, 0)

            @pl.loop(0, n_blocks // 2)
            def _(v):
                for bi in (0, 1):
                    b = 2 * v + bi
                    wait_block(bi)
                    for ci in range(_BLK):
                        if ci < 2:
                            guard = (v >= 1) if bi == 0 else None
                        else:
                            guard = None
                        issue(ci, bi, guard)
                        if ci == 0:
                            if bi == 0:
                                pl.when(v >= 1)(
                                    lambda: process(_BLK - 1, 1))
                            else:
                                process(_BLK - 1, 0)
                            if bi == 0:
                                load_block(b + 1, 1)
                            else:
                                pl.when(v < n_blocks // 2 - 1)(
                                    lambda: load_block(b + 1, 0))
                        else:
                            process(ci - 1, bi)

            process(_BLK - 1, 1)
            pltpu.make_async_copy(dummy, rows0, sem_s0).wait()
            pltpu.make_async_copy(dummy, rows1, sem_s1).wait()
            pltpu.sync_copy(cnt_v, degp_hbm.at[pl.ds(s * N, N)])

        @pl.when(c == 0)
        def _():
            do_relation(meta0_hbm, dst0_hbm, degp0_hbm)

        @pl.when(c == 1)
        def _():
            do_relation(meta1_hbm, dst1_hbm, degp1_hbm)

        plsc.subcore_barrier()

        def copy_out(agg_hbm):
            @pl.loop(s, n_row_blks, step=_NS)
            def _(g):
                r0 = g * row_blk
                pltpu.sync_copy(acc_sh.at[pl.ds(r0, row_blk)],
                                agg_hbm.at[pl.ds(r0, row_blk)])

        @pl.when(c == 0)
        def _():
            copy_out(agg0_hbm)

        @pl.when(c == 1)
        def _():
            copy_out(agg1_hbm)

    return k(h, meta0, dst0, meta1, dst1, zeros_acc)


_ROW_BLK = 2000


def _tc_finish(a0, a1, g0, g1, W0, b0, W1, b1, A1, ab1, A2):
    N, D = a0.shape
    B = _ROW_BLK
    nb = N // B
    hp = lax.Precision.HIGHEST

    def body1(a0_r, a1_r, g0_r, g1_r, w0_r, c0_r, w1_r, c1_r, am_r, ab_r, a2_r,
              x0_o, x1_o, s0_o, s1_o):
        i = pl.program_id(0)
        d0 = jnp.maximum(jnp.sum(g0_r[...], axis=1, keepdims=True), 1.0)
        d1 = jnp.maximum(jnp.sum(g1_r[...], axis=1, keepdims=True), 1.0)
        x0 = jnp.dot(a0_r[...], w0_r[...], precision=hp,
                     preferred_element_type=jnp.float32) / d0 + c0_r[...]
        x1 = jnp.dot(a1_r[...], w1_r[...], precision=hp,
                     preferred_element_type=jnp.float32) / d1 + c1_r[...]
        x0 = jnp.where(x0 > 0, x0, jnp.exp(jnp.minimum(x0, 0.0)) - 1.0)
        x1 = jnp.where(x1 > 0, x1, jnp.exp(jnp.minimum(x1, 0.0)) - 1.0)
        x0_o[...] = x0
        x1_o[...] = x1
        t0 = jnp.tanh(jnp.dot(x0, am_r[...], precision=hp,
                              preferred_element_type=jnp.float32) + ab_r[...])
        t1 = jnp.tanh(jnp.dot(x1, am_r[...], precision=hp,
                              preferred_element_type=jnp.float32) + ab_r[...])
        p0 = jnp.sum(t0 * a2_r[...])
        p1 = jnp.sum(t1 * a2_r[...])

        @pl.when(i == 0)
        def _():
            s0_o[...] = jnp.zeros_like(s0_o)
            s1_o[...] = jnp.zeros_like(s1_o)

        s0_o[...] += p0
        s1_o[...] += p1

    row_spec = pl.BlockSpec((B, D), lambda i: (i, 0))
    deg_spec = pl.BlockSpec((B, _NS), lambda i: (i, 0))
    full_spec = pl.BlockSpec((D, D), lambda i: (0, 0))
    vec_spec = pl.BlockSpec((1, D), lambda i: (0, 0))
    x0, x1, ssum0, ssum1 = pl.pallas_call(
        body1,
        grid=(nb,),
        in_specs=[row_spec, row_spec, deg_spec, deg_spec,
                  full_spec, vec_spec, full_spec, vec_spec,
                  full_spec, vec_spec, vec_spec],
        out_specs=[row_spec, row_spec,
                   pl.BlockSpec((1, D), lambda i: (0, 0)),
                   pl.BlockSpec((1, D), lambda i: (0, 0))],
        out_shape=[jax.ShapeDtypeStruct((N, D), jnp.float32),
                   jax.ShapeDtypeStruct((N, D), jnp.float32),
                   jax.ShapeDtypeStruct((1, D), jnp.float32),
                   jax.ShapeDtypeStruct((1, D), jnp.float32)],
    )(a0, a1, g0, g1, W0, b0, W1, b1, A1, ab1, A2)

    def body2(x0_r, x1_r, s0_r, s1_r, out_r):
        w0m = s0_r[0, 0] / N
        w1m = s1_r[0, 0] / N
        m = jnp.maximum(w0m, w1m)
        e0 = jnp.exp(w0m - m)
        e1 = jnp.exp(w1m - m)
        beta0 = e0 / (e0 + e1)
        beta1 = e1 / (e0 + e1)
        out_r[...] = beta0 * x0_r[...] + beta1 * x1_r[...]

    return pl.pallas_call(
        body2,
        grid=(nb,),
        in_specs=[row_spec, row_spec, vec_spec, vec_spec],
        out_specs=row_spec,
        out_shape=jax.ShapeDtypeStruct((N, D), jnp.float32),
    )(x0, x1, ssum0, ssum1)


def kernel(h, edge_index_r0, edge_index_r1, ew_r0, ew_r1,
           W0, b0, W1, b1, A1, ab1, A2):
    N, D = h.shape
    E = ew_r0.shape[0]
    zeros_acc = jnp.zeros((N, D), jnp.float32)
    meta0 = _pack_meta(edge_index_r0[0], ew_r0, edge_index_r0[1])
    meta1 = _pack_meta(edge_index_r1[0], ew_r1, edge_index_r1[1])
    agg0, agg1, degp0, degp1 = _sc_aggregate(
        h, meta0, edge_index_r0[1], meta1, edge_index_r1[1], zeros_acc, E)
    g0 = degp0.reshape(_NS, N).T
    g1 = degp1.reshape(_NS, N).T
    return _tc_finish(
        agg0, agg1, g0, g1,
        W0, b0.reshape(1, D), W1, b1.reshape(1, D),
        A1, ab1.reshape(1, -1), A2.reshape(1, -1),
    )
